# baseline (speedup 1.0000x reference)
import jax
import jax.numpy as jnp
from jax.experimental import pallas as pl
from jax.experimental.pallas import tpu as pltpu


def _adapter_body(idx_ref, x_ref, dw_ref, db_ref, uw_ref, o_ref, z_ref):
    dw = dw_ref[0, 0].astype(jnp.bfloat16)
    uw = uw_ref[0, 0].astype(jnp.bfloat16)
    db = db_ref[0, 0]
    T = 64
    n_t = x_ref.shape[1] // T
    zs = []
    for t in range(n_t):
        xt = x_ref[0, t * T:(t + 1) * T, :].astype(jnp.bfloat16)
        z = jnp.dot(xt, dw, preferred_element_type=jnp.float32) + db
        z = z * jax.nn.sigmoid(z)
        zs.append(z.astype(jnp.bfloat16))
    for t in range(n_t):
        o_ref[0, 0, t * T:(t + 1) * T, :] = jnp.dot(
            zs[t], uw, preferred_element_type=jnp.float32)


def kernel(x, expert_index, down_w, down_b, up_w):
    B, S, C = x.shape
    M, N, _, D = down_w.shape
    TS = 1024
    idx = expert_index.astype(jnp.int32)
    db4 = down_b.reshape(M, N, 1, D)

    grid = (M, B, S // TS)

    out = pl.pallas_call(
        _adapter_body,
        grid_spec=pltpu.PrefetchScalarGridSpec(
            num_scalar_prefetch=1,
            grid=grid,
            in_specs=[
                pl.BlockSpec((1, TS, C), lambda m, b, s, i: (b, s, 0)),
                pl.BlockSpec((1, 1, C, D), lambda m, b, s, i: (m, i[m, b], 0, 0)),
                pl.BlockSpec((1, 1, 1, D), lambda m, b, s, i: (m, i[m, b], 0, 0)),
                pl.BlockSpec((1, 1, D, C), lambda m, b, s, i: (m, i[m, b], 0, 0)),
            ],
            out_specs=pl.BlockSpec((1, 1, TS, C), lambda m, b, s, i: (m, b, s, 0)),
            scratch_shapes=[pltpu.VMEM((TS, C), jnp.float32)],
        ),
        out_shape=jax.ShapeDtypeStruct((M, B, S, C), x.dtype),
        compiler_params=pltpu.CompilerParams(
            dimension_semantics=("parallel", "parallel", "arbitrary"),
        ),
    )(idx, x, down_w, db4, up_w)
    return out


# speedup vs baseline: 1.2280x; 1.2280x over previous
import jax
import jax.numpy as jnp
from jax.experimental import pallas as pl
from jax.experimental.pallas import tpu as pltpu


def _adapter_body(idx_ref, x_ref, dw_ref, db_ref, uw_ref, o_ref, z_ref):
    dw = dw_ref[0, 0].astype(jnp.bfloat16)
    uw = uw_ref[0, 0].astype(jnp.bfloat16)
    db = db_ref[0, 0]
    T = 128
    n_t = x_ref.shape[1] // T
    zs = []
    for t in range(n_t):
        xt = x_ref[0, t * T:(t + 1) * T, :].astype(jnp.bfloat16)
        z = jnp.dot(xt, dw, preferred_element_type=jnp.float32) + db
        z = z * jax.nn.sigmoid(z)
        zs.append(z.astype(jnp.bfloat16))
    for t in range(n_t):
        o_ref[0, 0, t * T:(t + 1) * T, :] = jnp.dot(
            zs[t], uw, preferred_element_type=jnp.float32)


def kernel(x, expert_index, down_w, down_b, up_w):
    B, S, C = x.shape
    M, N, _, D = down_w.shape
    TS = 1024
    idx = expert_index.astype(jnp.int32)
    db4 = down_b.reshape(M, N, 1, D)

    grid = (M, B, S // TS)

    out = pl.pallas_call(
        _adapter_body,
        grid_spec=pltpu.PrefetchScalarGridSpec(
            num_scalar_prefetch=1,
            grid=grid,
            in_specs=[
                pl.BlockSpec((1, TS, C), lambda m, b, s, i: (b, s, 0)),
                pl.BlockSpec((1, 1, C, D), lambda m, b, s, i: (m, i[m, b], 0, 0)),
                pl.BlockSpec((1, 1, 1, D), lambda m, b, s, i: (m, i[m, b], 0, 0)),
                pl.BlockSpec((1, 1, D, C), lambda m, b, s, i: (m, i[m, b], 0, 0)),
            ],
            out_specs=pl.BlockSpec((1, 1, TS, C), lambda m, b, s, i: (m, b, s, 0)),
            scratch_shapes=[pltpu.VMEM((TS, C), jnp.float32)],
        ),
        out_shape=jax.ShapeDtypeStruct((M, B, S, C), x.dtype),
        compiler_params=pltpu.CompilerParams(
            dimension_semantics=("arbitrary", "arbitrary", "arbitrary"),
        ),
    )(idx, x, down_w, db4, up_w)
    return out


# speedup vs baseline: 1.2295x; 1.0013x over previous
import jax
import jax.numpy as jnp
from jax.experimental import pallas as pl
from jax.experimental.pallas import tpu as pltpu


def _adapter_body(idx_ref, x_ref, dw_ref, db_ref, uw_ref, o_ref):
    dw = dw_ref[0, 0].astype(jnp.bfloat16)
    uw = uw_ref[0, 0].astype(jnp.bfloat16)
    db = db_ref[0, 0]
    T = 128
    n_t = x_ref.shape[1] // T
    zs = []
    for t in range(n_t):
        xt = x_ref[0, t * T:(t + 1) * T, :].astype(jnp.bfloat16)
        z = jnp.dot(xt, dw, preferred_element_type=jnp.float32) + db
        z = z * jax.nn.sigmoid(z)
        zs.append(z.astype(jnp.bfloat16))
    for t in range(n_t):
        o_ref[0, 0, t * T:(t + 1) * T, :] = jnp.dot(
            zs[t], uw, preferred_element_type=jnp.float32)


def kernel(x, expert_index, down_w, down_b, up_w):
    B, S, C = x.shape
    M, N, _, D = down_w.shape
    TS = 1024
    idx = expert_index.astype(jnp.int32)
    db4 = down_b.reshape(M, N, 1, D)

    grid = (M, B, S // TS)

    out = pl.pallas_call(
        _adapter_body,
        grid_spec=pltpu.PrefetchScalarGridSpec(
            num_scalar_prefetch=1,
            grid=grid,
            in_specs=[
                pl.BlockSpec((1, TS, C), lambda m, b, s, i: (b, s, 0)),
                pl.BlockSpec((1, 1, C, D), lambda m, b, s, i: (m, i[m, b], 0, 0)),
                pl.BlockSpec((1, 1, 1, D), lambda m, b, s, i: (m, i[m, b], 0, 0)),
                pl.BlockSpec((1, 1, D, C), lambda m, b, s, i: (m, i[m, b], 0, 0)),
            ],
            out_specs=pl.BlockSpec((1, 1, TS, C), lambda m, b, s, i: (m, b, s, 0)),
        ),
        out_shape=jax.ShapeDtypeStruct((M, B, S, C), x.dtype),
        compiler_params=pltpu.CompilerParams(
            dimension_semantics=("parallel", "parallel", "arbitrary"),
        ),
    )(idx, x, down_w, db4, up_w)
    return out
